# baseline (speedup 1.0000x reference)
import jax
import jax.numpy as jnp
from jax import lax
from jax.experimental import pallas as pl
from jax.experimental.pallas import tpu as pltpu
from jax.experimental.pallas import tpu_sc as plsc

N_NODES = 10000
N_PAIRS = 640000
MAXD = 20

NC = 2
NS = 16
NW = NC * NS
L = 16

PER_TILE = N_PAIRS // NW
CHUNK = 80
NCHUNK = PER_TILE // CHUNK
B_PAD = 32
HPRIME = 32749


def _body(lin_hbm, idx_hbm, b_hbm, out_hbm, lin_v, idx_v, val2, lin2, b_v, sem):
    cid = lax.axis_index("c")
    sid = lax.axis_index("s")
    wid = sid * NC + cid
    base = wid * PER_TILE

    pltpu.sync_copy(b_hbm, b_v)
    pltpu.sync_copy(lin_hbm.at[pl.ds(base, PER_TILE)], lin_v)
    pltpu.sync_copy(idx_hbm.at[pl.ds(base, PER_TILE)], idx_v)

    @pl.loop(0, NCHUNK)
    def _compute(j):
        for t in range(CHUNK // L):
            sl = pl.ds(j * CHUNK + t * L, L)
            val = plsc.load_gather(b_v, [idx_v[sl]])
            val2[j, pl.ds(t * L, L)] = val
            lin2[j, pl.ds(t * L, L)] = lin_v[sl]

    @pl.loop(0, NCHUNK)
    def _scatter(j):
        pltpu.sync_copy(val2.at[j], out_hbm.at[lin2.at[j]])


@jax.jit
def _run(lin, win_idx, b_pad):
    out_ref = jax.new_ref(jnp.zeros((N_NODES * N_NODES,), jnp.float32))
    mesh = plsc.VectorSubcoreMesh(
        core_axis_name="c", subcore_axis_name="s", num_cores=NC, num_subcores=NS
    )
    scatter = pl.kernel(
        _body,
        out_type=(),
        mesh=mesh,
        compiler_params=pltpu.CompilerParams(needs_layout_passes=False),
        scratch_types=[
            pltpu.VMEM((PER_TILE,), jnp.int32),
            pltpu.VMEM((PER_TILE,), jnp.int32),
            pltpu.VMEM((NCHUNK, CHUNK), jnp.float32),
            pltpu.VMEM((NCHUNK, CHUNK), jnp.int32),
            pltpu.VMEM((B_PAD,), jnp.float32),
            pltpu.SemaphoreType.DMA,
        ],
    )
    scatter(lin, win_idx, b_pad, out_ref)
    return jax.freeze(out_ref).reshape(N_NODES, N_NODES)


def kernel(x, distances, distances_index, b):
    del x
    lin = distances_index[0] * N_NODES + distances_index[1]
    idxb = jnp.minimum(distances, float(MAXD - 1)).astype(jnp.int32)
    pos = jnp.arange(N_PAIRS, dtype=jnp.int32)

    h = ((pos % HPRIME) + 1).astype(jnp.int16)
    marks = jnp.zeros((N_NODES * N_NODES,), jnp.int16).at[lin].set(h)
    win = marks[lin] == h

    sk, sp = lax.sort((lin, pos), num_keys=1)
    seg = jnp.cumsum(jnp.concatenate(
        [jnp.zeros((1,), jnp.int32), (sk[1:] != sk[:-1]).astype(jnp.int32)]))
    cand = jnp.where(win[sp], idxb[sp], -1)
    segmax = jnp.full((N_PAIRS,), -1, jnp.int32).at[seg].max(cand)
    win_idx = jnp.zeros((N_PAIRS,), jnp.int32).at[sp].set(segmax[seg])

    b_pad = jnp.zeros((B_PAD,), b.dtype).at[:MAXD].set(b)
    return _run(lin, win_idx, b_pad)
